# baseline (speedup 1.0000x reference)
import functools

import jax
import jax.numpy as jnp
from jax import lax
from jax.experimental import pallas as pl
from jax.experimental.pallas import tpu as pltpu
from jax.experimental.pallas import tpu_sc as plsc


def _make_sc_gather(B, V_u, V_i, DIM):
    info = plsc.get_sparse_core_info()
    NC, NS = info.num_cores, info.num_subcores
    NW = NC * NS
    assert B % NW == 0 and (B // NW) % 8 == 0
    bpw = B // NW
    mesh = plsc.VectorSubcoreMesh(core_axis_name="c", subcore_axis_name="s")

    @functools.partial(
        pl.kernel,
        mesh=mesh,
        compiler_params=pltpu.CompilerParams(use_tc_tiling_on_sc=False),
        out_type=[
            jax.ShapeDtypeStruct((B, DIM), jnp.float32),
            jax.ShapeDtypeStruct((B, DIM), jnp.float32),
            jax.ShapeDtypeStruct((B,), jnp.float32),
        ],
        scratch_types=[
            pltpu.VMEM((bpw,), jnp.int32),
            pltpu.VMEM((bpw,), jnp.int32),
            pltpu.VMEM((bpw, DIM), jnp.float32),
            pltpu.VMEM((bpw, DIM), jnp.float32),
            pltpu.VMEM((bpw,), jnp.float32),
            pltpu.SemaphoreType.DMA,
            pltpu.SemaphoreType.DMA,
            pltpu.SemaphoreType.DMA,
        ],
    )
    def sc_gather(user_h, item_h, gu_h, gi_h, bi_h, gu_o, gi_o, bi_o,
                  uidx_v, iidx_v, gu_v, gi_v, bi_v, s0, s1, s2):
        wid = lax.axis_index("s") * NC + lax.axis_index("c")
        base = wid * bpw
        pltpu.sync_copy(user_h.at[pl.ds(base, bpw)], uidx_v)
        pltpu.sync_copy(item_h.at[pl.ds(base, bpw)], iidx_v)
        cu = pltpu.async_copy(gu_h.at[uidx_v], gu_v, s0)
        ci = pltpu.async_copy(gi_h.at[iidx_v], gi_v, s1)
        cb = pltpu.async_copy(bi_h.at[iidx_v], bi_v, s2)
        cu.wait()
        pltpu.sync_copy(gu_v, gu_o.at[pl.ds(base, bpw)])
        ci.wait()
        pltpu.sync_copy(gi_v, gi_o.at[pl.ds(base, bpw)])
        cb.wait()
        pltpu.sync_copy(bi_v, bi_o.at[pl.ds(base, bpw)])

    return sc_gather


def _score_body(gu_ref, gi_ref, beta_ref, out_ref):
    d = gu_ref[...] - gi_ref[...]
    l2 = jnp.sum(d * d, axis=1, keepdims=True)
    out_ref[...] = beta_ref[...] - l2


def _make_tc_score(B, DIM, rows):
    grid = (B // rows,)
    return pl.pallas_call(
        _score_body,
        grid=grid,
        in_specs=[
            pl.BlockSpec((rows, DIM), lambda i: (i, 0)),
            pl.BlockSpec((rows, DIM), lambda i: (i, 0)),
            pl.BlockSpec((1, B), lambda i: (0, 0)),
        ],
        out_specs=pl.BlockSpec((rows, B), lambda i: (i, 0)),
        out_shape=jax.ShapeDtypeStruct((B, B), jnp.float32),
    )


def kernel(user, item, Gu, Gi, Bi):
    B = user.shape[0]
    DIM = Gu.shape[1]
    gather = _make_sc_gather(B, Gu.shape[0], Gi.shape[0], DIM)
    gamma_u, gamma_i, beta = gather(user, item, Gu, Gi, Bi.reshape(-1))
    score = _make_tc_score(B, DIM, 256)(gamma_u, gamma_i, beta.reshape(1, B))
    return (score, beta, gamma_u, gamma_i)


# speedup vs baseline: 5.9094x; 5.9094x over previous
import functools

import jax
import jax.numpy as jnp
from jax import lax
from jax.experimental import pallas as pl
from jax.experimental.pallas import tpu as pltpu
from jax.experimental.pallas import tpu_sc as plsc

_RING = 4
_LANE = 16


def _make_sc_gather(B, V, DIM):
    info = plsc.get_sparse_core_info()
    NC, NS = info.num_cores, info.num_subcores
    NW = NC * NS
    assert B % NW == 0 and (B // NW) % _LANE == 0
    bpw = B // NW
    nt = bpw // _LANE
    mesh = plsc.VectorSubcoreMesh(core_axis_name="c", subcore_axis_name="s")

    @functools.partial(
        pl.kernel,
        mesh=mesh,
        compiler_params=pltpu.CompilerParams(
            use_tc_tiling_on_sc=True, needs_layout_passes=False
        ),
        out_type=[
            jax.ShapeDtypeStruct((B, DIM), jnp.float32),
            jax.ShapeDtypeStruct((B, DIM), jnp.float32),
            jax.ShapeDtypeStruct((B,), jnp.float32),
        ],
        scratch_types=[
            pltpu.VMEM((bpw,), jnp.int32),
            pltpu.VMEM((bpw,), jnp.int32),
            pltpu.VMEM((_RING, DIM, 128), jnp.float32),
            pltpu.VMEM((_RING, DIM, 128), jnp.float32),
            pltpu.VMEM((bpw, DIM), jnp.float32),
            pltpu.VMEM((bpw, DIM), jnp.float32),
            pltpu.VMEM((bpw,), jnp.float32),
            [pltpu.SemaphoreType.DMA] * _RING,
            [pltpu.SemaphoreType.DMA] * _RING,
            pltpu.SemaphoreType.DMA,
        ],
    )
    def sc_gather(user_h, item_h, gut_h, git_h, bi_h, gu_o, gi_o, bi_o,
                  uidx_v, iidx_v, blk_u, blk_i, rows_u, rows_i, bi_v,
                  sems_u, sems_i, sem_bi):
        wid = lax.axis_index("s") * NC + lax.axis_index("c")
        base = wid * bpw
        pltpu.sync_copy(user_h.at[pl.ds(base, bpw)], uidx_v)
        pltpu.sync_copy(item_h.at[pl.ds(base, bpw)], iidx_v)

        cb = pltpu.async_copy(bi_h.at[iidx_v], bi_v, sem_bi)

        def col_of(i):
            return pl.multiple_of((i >> 7) * 128, 128)

        def fetch(tab, blk, sems, i, slot):
            pltpu.async_copy(
                tab.at[:, pl.ds(col_of(i), 128)], blk.at[slot], sems[slot]
            )

        def drain(tab, blk, sems, slot):
            pltpu.make_async_copy(
                tab.at[:, pl.ds(0, 128)], blk.at[slot], sems[slot]
            ).wait()

        def extract(tab, blk, sems, rows, i, b, slot):
            drain(tab, blk, sems, slot)
            w = jnp.full((_LANE,), 0, jnp.int32) + (i & 127)
            sl = jnp.full((_LANE,), slot, jnp.int32)
            for jt in range(DIM // _LANE):
                j16 = lax.broadcasted_iota(jnp.int32, (_LANE,), 0) + jt * _LANE
                vals = plsc.load_gather(blk, [sl, j16, w])
                rows[b, pl.ds(jt * _LANE, _LANE)] = vals

        idx16u0 = uidx_v[pl.ds(0, _LANE)]
        idx16i0 = iidx_v[pl.ds(0, _LANE)]
        for l in range(_RING):
            fetch(gut_h, blk_u, sems_u, idx16u0[l], l)
            fetch(git_h, blk_i, sems_i, idx16i0[l], l)

        vmax = jnp.int32(V - 1)

        def t_body(t, carry):
            idx16u = uidx_v[pl.ds(t * _LANE, _LANE)]
            idx16i = iidx_v[pl.ds(t * _LANE, _LANE)]
            tn = jnp.minimum(t + 1, nt - 1)
            nx_u = jnp.minimum(uidx_v[pl.ds(tn * _LANE, _LANE)], vmax)
            nx_i = jnp.minimum(iidx_v[pl.ds(tn * _LANE, _LANE)], vmax)
            for l in range(_LANE):
                b = t * _LANE + l
                slot = l % _RING
                extract(gut_h, blk_u, sems_u, rows_u, idx16u[l], b, slot)
                extract(git_h, blk_i, sems_i, rows_i, idx16i[l], b, slot)
                if l + _RING < _LANE:
                    fu, fi = idx16u[l + _RING], idx16i[l + _RING]
                else:
                    fu, fi = nx_u[l + _RING - _LANE], nx_i[l + _RING - _LANE]
                fetch(gut_h, blk_u, sems_u, fu, slot)
                fetch(git_h, blk_i, sems_i, fi, slot)
            return carry

        lax.fori_loop(0, nt, t_body, 0)

        for l in range(_RING):
            drain(gut_h, blk_u, sems_u, l)
            drain(git_h, blk_i, sems_i, l)

        cb.wait()
        pltpu.sync_copy(rows_u, gu_o.at[pl.ds(base, bpw)])
        pltpu.sync_copy(rows_i, gi_o.at[pl.ds(base, bpw)])
        pltpu.sync_copy(bi_v, bi_o.at[pl.ds(base, bpw)])

    return sc_gather


def _score_body(gu_ref, gi_ref, beta_ref, out_ref):
    d = gu_ref[...] - gi_ref[...]
    l2 = jnp.sum(d * d, axis=1, keepdims=True)
    out_ref[...] = beta_ref[...] - l2


def _make_tc_score(B, DIM, rows):
    return pl.pallas_call(
        _score_body,
        grid=(B // rows,),
        in_specs=[
            pl.BlockSpec((rows, DIM), lambda i: (i, 0)),
            pl.BlockSpec((rows, DIM), lambda i: (i, 0)),
            pl.BlockSpec((1, B), lambda i: (0, 0)),
        ],
        out_specs=pl.BlockSpec((rows, B), lambda i: (i, 0)),
        out_shape=jax.ShapeDtypeStruct((B, B), jnp.float32),
    )


def kernel(user, item, Gu, Gi, Bi):
    B = user.shape[0]
    V, DIM = Gu.shape
    gather = _make_sc_gather(B, V, DIM)
    gamma_u, gamma_i, beta = gather(user, item, Gu.T, Gi.T, Bi.reshape(-1))
    score = _make_tc_score(B, DIM, 256)(gamma_u, gamma_i, beta.reshape(1, B))
    return (score, beta, gamma_u, gamma_i)


# speedup vs baseline: 6.8854x; 1.1651x over previous
import functools

import jax
import jax.numpy as jnp
from jax import lax
from jax.experimental import pallas as pl
from jax.experimental.pallas import tpu as pltpu
from jax.experimental.pallas import tpu_sc as plsc

_RING = 4
_LANE = 16
_STAGES = 2
_TC_ROWS = 256


def _sc_info():
    info = plsc.get_sparse_core_info()
    return info.num_cores, info.num_subcores


def _make_beta_gather(B):
    NC, NS = _sc_info()
    NW = NC * NS
    bpw = B // NW
    mesh = plsc.VectorSubcoreMesh(core_axis_name="c", subcore_axis_name="s")

    @functools.partial(
        pl.kernel,
        mesh=mesh,
        compiler_params=pltpu.CompilerParams(
            use_tc_tiling_on_sc=True, needs_layout_passes=False
        ),
        out_type=[jax.ShapeDtypeStruct((B,), jnp.float32)],
        scratch_types=[
            pltpu.VMEM((bpw,), jnp.int32),
            pltpu.VMEM((bpw,), jnp.float32),
            pltpu.SemaphoreType.DMA,
        ],
    )
    def beta_gather(item_h, bi_h, bi_o, iidx_v, bi_v, sem):
        wid = lax.axis_index("s") * NC + lax.axis_index("c")
        base = wid * bpw
        pltpu.sync_copy(item_h.at[pl.ds(base, bpw)], iidx_v)
        pltpu.async_copy(bi_h.at[iidx_v], bi_v, sem).wait()
        pltpu.sync_copy(bi_v, bi_o.at[pl.ds(base, bpw)])

    return beta_gather


def _make_sc_gather(R, V, DIM):
    NC, NS = _sc_info()
    NW = NC * NS
    assert R % NW == 0 and (R // NW) % _LANE == 0
    bpw = R // NW
    nt = bpw // _LANE
    mesh = plsc.VectorSubcoreMesh(core_axis_name="c", subcore_axis_name="s")

    @functools.partial(
        pl.kernel,
        mesh=mesh,
        compiler_params=pltpu.CompilerParams(
            use_tc_tiling_on_sc=True, needs_layout_passes=False
        ),
        out_type=[
            jax.ShapeDtypeStruct((R, DIM), jnp.float32),
            jax.ShapeDtypeStruct((R, DIM), jnp.float32),
        ],
        scratch_types=[
            pltpu.VMEM((bpw,), jnp.int32),
            pltpu.VMEM((bpw,), jnp.int32),
            pltpu.VMEM((_RING, DIM, 128), jnp.float32),
            pltpu.VMEM((_RING, DIM, 128), jnp.float32),
            pltpu.VMEM((bpw, DIM), jnp.float32),
            pltpu.VMEM((bpw, DIM), jnp.float32),
            [pltpu.SemaphoreType.DMA] * _RING,
            [pltpu.SemaphoreType.DMA] * _RING,
        ],
    )
    def sc_gather(user_h, item_h, gut_h, git_h, gu_o, gi_o,
                  uidx_v, iidx_v, blk_u, blk_i, rows_u, rows_i,
                  sems_u, sems_i):
        wid = lax.axis_index("s") * NC + lax.axis_index("c")
        base = wid * bpw
        pltpu.sync_copy(user_h.at[pl.ds(base, bpw)], uidx_v)
        pltpu.sync_copy(item_h.at[pl.ds(base, bpw)], iidx_v)

        def col_of(i):
            return pl.multiple_of((i >> 7) * 128, 128)

        def fetch(tab, blk, sems, i, slot):
            pltpu.async_copy(
                tab.at[:, pl.ds(col_of(i), 128)], blk.at[slot], sems[slot]
            )

        def drain(tab, blk, sems, slot):
            pltpu.make_async_copy(
                tab.at[:, pl.ds(0, 128)], blk.at[slot], sems[slot]
            ).wait()

        def extract(tab, blk, sems, rows, i, b, slot):
            drain(tab, blk, sems, slot)
            w = jnp.full((_LANE,), 0, jnp.int32) + (i & 127)
            sl = jnp.full((_LANE,), slot, jnp.int32)
            for jt in range(DIM // _LANE):
                j16 = lax.broadcasted_iota(jnp.int32, (_LANE,), 0) + jt * _LANE
                vals = plsc.load_gather(blk, [sl, j16, w])
                rows[b, pl.ds(jt * _LANE, _LANE)] = vals

        idx16u0 = uidx_v[pl.ds(0, _LANE)]
        idx16i0 = iidx_v[pl.ds(0, _LANE)]
        for l in range(_RING):
            fetch(gut_h, blk_u, sems_u, idx16u0[l], l)
            fetch(git_h, blk_i, sems_i, idx16i0[l], l)

        vmax = jnp.int32(V - 1)

        def t_body(t, carry):
            idx16u = uidx_v[pl.ds(t * _LANE, _LANE)]
            idx16i = iidx_v[pl.ds(t * _LANE, _LANE)]
            tn = jnp.minimum(t + 1, nt - 1)
            nx_u = jnp.minimum(uidx_v[pl.ds(tn * _LANE, _LANE)], vmax)
            nx_i = jnp.minimum(iidx_v[pl.ds(tn * _LANE, _LANE)], vmax)
            for l in range(_LANE):
                b = t * _LANE + l
                slot = l % _RING
                extract(gut_h, blk_u, sems_u, rows_u, idx16u[l], b, slot)
                extract(git_h, blk_i, sems_i, rows_i, idx16i[l], b, slot)
                if l + _RING < _LANE:
                    fu, fi = idx16u[l + _RING], idx16i[l + _RING]
                else:
                    fu, fi = nx_u[l + _RING - _LANE], nx_i[l + _RING - _LANE]
                fetch(gut_h, blk_u, sems_u, fu, slot)
                fetch(git_h, blk_i, sems_i, fi, slot)
            return carry

        lax.fori_loop(0, nt, t_body, 0)

        for l in range(_RING):
            drain(gut_h, blk_u, sems_u, l)
            drain(git_h, blk_i, sems_i, l)

        pltpu.sync_copy(rows_u, gu_o.at[pl.ds(base, bpw)])
        pltpu.sync_copy(rows_i, gi_o.at[pl.ds(base, bpw)])

    return sc_gather


def _score_body(gu_ref, gi_ref, beta_ref, out_ref):
    d = gu_ref[...] - gi_ref[...]
    l2 = jnp.sum(d * d, axis=1, keepdims=True)
    out_ref[...] = beta_ref[...] - l2


def _score_body_acc(gu_ref, gi_ref, beta_ref, _acc_ref, out_ref):
    _score_body(gu_ref, gi_ref, beta_ref, out_ref)


def _make_tc_score(B, DIM, rows, R, stage, first):
    blk0 = stage * (R // rows)
    in_specs = [
        pl.BlockSpec((rows, DIM), lambda i: (i, 0)),
        pl.BlockSpec((rows, DIM), lambda i: (i, 0)),
        pl.BlockSpec((1, B), lambda i: (0, 0)),
    ]
    kwargs = {}
    body = _score_body
    if not first:
        in_specs.append(pl.BlockSpec((8, 128), lambda i: (0, 0)))
        kwargs["input_output_aliases"] = {3: 0}
        body = _score_body_acc
    return pl.pallas_call(
        body,
        grid=(R // rows,),
        in_specs=in_specs,
        out_specs=pl.BlockSpec((rows, B), lambda i: (blk0 + i, 0)),
        out_shape=jax.ShapeDtypeStruct((B, B), jnp.float32),
        **kwargs,
    )


def kernel(user, item, Gu, Gi, Bi):
    B = user.shape[0]
    V, DIM = Gu.shape
    R = B // _STAGES
    GuT, GiT = Gu.T, Gi.T

    (beta,) = _make_beta_gather(B)(item, Bi.reshape(-1))
    beta_row = beta.reshape(1, B)

    gather = _make_sc_gather(R, V, DIM)
    gus, gis, score = [], [], None
    for h in range(_STAGES):
        sl = slice(h * R, (h + 1) * R)
        gu_h, gi_h = gather(user[sl], item[sl], GuT, GiT)
        gus.append(gu_h)
        gis.append(gi_h)
        tc = _make_tc_score(B, DIM, _TC_ROWS, R, h, score is None)
        if score is None:
            score = tc(gu_h, gi_h, beta_row)
        else:
            score = tc(gu_h, gi_h, beta_row, score)

    gamma_u = jnp.concatenate(gus, axis=0)
    gamma_i = jnp.concatenate(gis, axis=0)
    return (score, beta, gamma_u, gamma_i)
